# baseline (speedup 1.0000x reference)
import jax
import jax.numpy as jnp
import numpy as np
from jax import lax
from jax.experimental import pallas as pl
from jax.experimental.pallas import tpu as pltpu

INHIBITION_INCREASE = 5.0
DECAY_RATE = 100.0
DECAY_SIGMA = 5.0
DT = 0.001
DT_SQRT = float(np.sqrt(DT))
_TINY = float(np.finfo(np.float32).tiny)

_U32 = np.uint32
_ROTS = ((13, 15, 26, 6), (17, 29, 16, 24))


def _np_threefry2x32(k1, k2, x0, x1):
    def rotl(x, r):
        return (x << _U32(r)) | (x >> _U32(32 - r))
    ks = [_U32(k1), _U32(k2), _U32(k1) ^ _U32(k2) ^ _U32(0x1BD11BDA)]
    x = [(x0 + ks[0]).astype(_U32), (x1 + ks[1]).astype(_U32)]
    for i in range(5):
        for r in _ROTS[i % 2]:
            x[0] = (x[0] + x[1]).astype(_U32)
            x[1] = rotl(x[1], r) ^ x[0]
        x[0] = (x[0] + ks[(i + 1) % 3]).astype(_U32)
        x[1] = (x[1] + ks[(i + 2) % 3] + _U32(i + 1)).astype(_U32)
    return x


def _np_split3(k1, k2):
    b1, b2 = _np_threefry2x32(k1, k2, np.zeros(3, _U32), np.arange(3, dtype=_U32))
    return [(int(b1[i]), int(b2[i])) for i in range(3)]


_KN, _KU, _KC = _np_split3(0, 42)


def _tf_bits(k1, k2, x1):
    ks0 = jnp.uint32(k1)
    ks1 = jnp.uint32(k2)
    ks2 = jnp.uint32(k1 ^ k2 ^ 0x1BD11BDA)
    ks = (ks0, ks1, ks2)
    x0 = jnp.broadcast_to(ks0, x1.shape)
    x1 = x1 + ks1
    for i in range(5):
        for r in _ROTS[i % 2]:
            x0 = x0 + x1
            x1 = ((x1 << r) | (x1 >> (32 - r))) ^ x0
        x0 = x0 + ks[(i + 1) % 3]
        x1 = x1 + ks[(i + 2) % 3] + jnp.uint32(i + 1)
    return x0 ^ x1


def _spike_body(x_ref, inhp_ref, rv_ref, out_ref, inh_ref):
    br, w = x_ref.shape
    b = pl.program_id(0)

    inh = inhp_ref[...]
    rates = jnp.clip(jnp.exp(x_ref[...] - inh), 1e-20, 1e20)
    logit = jnp.log(rates)
    total = jnp.sum(rates, axis=1, keepdims=True)

    cols = lax.broadcasted_iota(jnp.int32, (br, w), 1)
    rowbase = (lax.broadcasted_iota(jnp.int32, (br, 1), 0) + b * br) * w
    flat = (rowbase + cols).astype(jnp.uint32)

    bits = _tf_bits(_KC[0], _KC[1], flat)
    fb = (bits >> 9) | jnp.uint32(0x3F800000)
    frac = lax.bitcast_convert_type(fb, jnp.float32) - 1.0
    u = jnp.maximum(frac, jnp.float32(_TINY))
    score = logit - jnp.log(-jnp.log(u))

    m = jnp.max(score, axis=1, keepdims=True)
    idx = jnp.min(jnp.where(score == m, cols, jnp.int32(2**31 - 1)),
                  axis=1, keepdims=True)
    spike = jnp.where(rv_ref[...] < DT * total, 1.0, 0.0).astype(jnp.float32)

    out_ref[...] = jnp.where(cols == idx, spike, 0.0)
    inh_ref[...] = inh + spike * INHIBITION_INCREASE


def kernel(inputs, inhibition):
    rows, w = inputs.shape
    dtype = inputs.dtype

    key = jax.random.key(42)
    kn, ku, _ = jax.random.split(key, 3)
    noise = jax.random.normal(kn, inhibition.shape, dtype=inhibition.dtype)
    inh_pre = (1.0 - DECAY_RATE * DT) * inhibition + DECAY_SIGMA * DT_SQRT * noise
    rand_val = jax.random.uniform(ku, (rows, 1), dtype=dtype)

    br = 8
    grid = (rows // br,)
    out_spikes, inh_out = pl.pallas_call(
        _spike_body,
        grid=grid,
        in_specs=[
            pl.BlockSpec((br, w), lambda i: (i, 0)),
            pl.BlockSpec((br, 1), lambda i: (i, 0)),
            pl.BlockSpec((br, 1), lambda i: (i, 0)),
        ],
        out_specs=[
            pl.BlockSpec((br, w), lambda i: (i, 0)),
            pl.BlockSpec((br, 1), lambda i: (i, 0)),
        ],
        out_shape=[
            jax.ShapeDtypeStruct((rows, w), dtype),
            jax.ShapeDtypeStruct((rows, 1), dtype),
        ],
        compiler_params=pltpu.CompilerParams(
            dimension_semantics=("parallel",),
        ),
    )(inputs, inh_pre, rand_val)
    return (out_spikes, inh_out)
